# Initial kernel scaffold; baseline (speedup 1.0000x reference)
import functools

import jax
import jax.numpy as jnp
from jax import lax
from jax.experimental import pallas as pl
from jax.experimental.pallas import tpu as pltpu
from jax.experimental.pallas import tpu_sc as plsc

N = 10000
E = 320000
D = 128

NC = 2
NS = 16
NW = NC * NS
EPW = E // NW
K = 80
NCHUNK = EPW // K
RPT = N // NS
LANES = 16

_mesh = plsc.VectorSubcoreMesh(core_axis_name="c", subcore_axis_name="s")


@functools.partial(
    pl.kernel,
    out_type=jax.ShapeDtypeStruct((NC, 2 * N, LANES), jnp.float32),
    mesh=_mesh,
    scratch_types=[
        pltpu.VMEM((NCHUNK, K), jnp.int32),
        pltpu.VMEM((NCHUNK, K), jnp.int32),
        pltpu.VMEM((K, LANES), jnp.float32),
        pltpu.VMEM((125, LANES), jnp.float32),
        pltpu.VMEM_SHARED((2 * N, LANES), jnp.float32),
    ],
)
def _sc_degrees(src_hbm, dst_hbm, out_hbm, sbuf, dbuf, ones_v, zero_v, acc):
    core = lax.axis_index("c")
    sid = lax.axis_index("s")
    wid = sid * NC + core

    @pl.loop(0, K)
    def _(i):
        ones_v[i, pl.ds(0, LANES)] = jnp.full((LANES,), 1.0, jnp.float32)

    @pl.loop(0, 125)
    def _(i):
        zero_v[i, pl.ds(0, LANES)] = jnp.zeros((LANES,), jnp.float32)

    @pl.loop(0, 10)
    def _(k):
        pltpu.sync_copy(zero_v, acc.at[pl.ds(sid * 1250 + k * 125, 125)])

    plsc.subcore_barrier()

    pltpu.sync_copy(src_hbm.at[wid], sbuf)
    pltpu.sync_copy(dst_hbm.at[wid], dbuf)

    @pl.loop(0, NCHUNK)
    def _(c):
        @pl.loop(0, K // LANES)
        def _(k):
            v = dbuf[c, pl.ds(k * LANES, LANES)]
            dbuf[c, pl.ds(k * LANES, LANES)] = v + N

    @pl.loop(0, NCHUNK)
    def _(c):
        pltpu.sync_copy(ones_v, acc.at[sbuf.at[c]], add=True)
        pltpu.sync_copy(ones_v, acc.at[dbuf.at[c]], add=True)

    plsc.subcore_barrier()
    pltpu.sync_copy(acc.at[pl.ds(sid * 1250, 1250)],
                    out_hbm.at[core, pl.ds(sid * 1250, 1250)])


@functools.partial(
    pl.kernel,
    out_type=jax.ShapeDtypeStruct((NC, N, D), jnp.float32),
    mesh=_mesh,
    scratch_types=[
        pltpu.VMEM((NCHUNK, K), jnp.int32),
        pltpu.VMEM((NCHUNK, K), jnp.int32),
        pltpu.VMEM((NCHUNK, K), jnp.float32),
        pltpu.VMEM((K, D), jnp.float32),
        pltpu.VMEM_SHARED((N, D), jnp.float32),
        pltpu.SemaphoreType.DMA,
    ],
)
def _sc_msgpass(hs_hbm, src_hbm, dst_hbm, ew_hbm, out_hbm,
                sbuf, dbuf, wbuf, rows, acc, sem):
    core = lax.axis_index("c")
    sid = lax.axis_index("s")
    wid = sid * NC + core

    @pl.loop(0, K)
    def _(i):
        for j in range(D // LANES):
            rows[i, pl.ds(j * LANES, LANES)] = jnp.zeros((LANES,), jnp.float32)

    @pl.loop(0, RPT // K)
    def _(k):
        pltpu.sync_copy(rows, acc.at[pl.ds(sid * RPT + k * K, K)])

    rem = RPT % K
    pltpu.sync_copy(rows.at[pl.ds(0, rem)],
                    acc.at[pl.ds(sid * RPT + (RPT // K) * K, rem)])

    plsc.subcore_barrier()

    pltpu.sync_copy(src_hbm.at[wid], sbuf)
    pltpu.sync_copy(dst_hbm.at[wid], dbuf)
    pltpu.sync_copy(ew_hbm.at[wid], wbuf)

    @pl.loop(0, NCHUNK)
    def _(c):
        pltpu.async_copy(hs_hbm.at[sbuf.at[c]], rows, sem).wait()

        @pl.loop(0, K)
        def _(i):
            s = wbuf[c, i]
            bv = jnp.broadcast_to(s, (LANES,))
            for j in range(D // LANES):
                slc = (i, pl.ds(j * LANES, LANES))
                rows[slc] = rows[slc] * bv

        pltpu.sync_copy(rows, acc.at[dbuf.at[c]], add=True)

    plsc.subcore_barrier()
    pltpu.sync_copy(acc.at[pl.ds(sid * RPT, RPT)],
                    out_hbm.at[core, pl.ds(sid * RPT, RPT)])


def _tc_prep_body(ds0, ds1, dd0, dd1, x, dinv_o, dinv_i, hs1):
    deg_o = jnp.maximum(ds0[...] + ds1[...], 1.0)
    deg_i = jnp.maximum(dd0[...] + dd1[...], 1.0)
    io = lax.rsqrt(deg_o)
    ii = lax.rsqrt(deg_i)
    dinv_o[...] = io
    dinv_i[...] = ii
    hs1[...] = x[...] * io


def _tc_prep(ds0, ds1, dd0, dd1, x):
    return pl.pallas_call(
        _tc_prep_body,
        out_shape=(
            jax.ShapeDtypeStruct((N, 1), jnp.float32),
            jax.ShapeDtypeStruct((N, 1), jnp.float32),
            jax.ShapeDtypeStruct((N, D), jnp.float32),
        ),
    )(ds0, ds1, dd0, dd1, x)


def _tc_layer_body(a, dinv_i, dinv_o, W, b, gamma, beta, out, *, elu, rescale):
    t = (a[0] + a[1]) * dinv_i[...]
    y = jnp.dot(t, W[...], preferred_element_type=jnp.float32) + b[...]
    mean = jnp.mean(y, axis=0, keepdims=True)
    var = jnp.mean((y - mean) ** 2, axis=0, keepdims=True)
    z = (y - mean) * lax.rsqrt(var + 1e-5) * gamma[...] + beta[...]
    if elu:
        z = jnp.where(z > 0, z, jnp.expm1(z))
    if rescale:
        z = z * dinv_o[...]
    out[...] = z


def _tc_layer(a, dinv_i, dinv_o, W, b, gamma, beta, *, elu, rescale):
    body = functools.partial(_tc_layer_body, elu=elu, rescale=rescale)
    return pl.pallas_call(
        body,
        out_shape=jax.ShapeDtypeStruct((N, D), jnp.float32),
    )(a, dinv_i, dinv_o, W, b, gamma, beta)


def kernel(node_weight, edge_index, edge_weight, W1, b1, W2, b2, W3, b3,
           gamma, beta):
    src = edge_index[0].reshape(NW, NCHUNK, K)
    dst = edge_index[1].reshape(NW, NCHUNK, K)
    ew = edge_weight.reshape(NW, NCHUNK, K)

    degs = _sc_degrees(src, dst)
    ds0 = degs[0, :N, 0:1]
    ds1 = degs[1, :N, 0:1]
    dd0 = degs[0, N:, 0:1]
    dd1 = degs[1, N:, 0:1]

    dinv_o, dinv_i, hs = _tc_prep(ds0, ds1, dd0, dd1, node_weight)

    a = _sc_msgpass(hs, src, dst, ew)
    hs = _tc_layer(a, dinv_i, dinv_o, W1, b1.reshape(1, D),
                   gamma.reshape(1, D), beta.reshape(1, D),
                   elu=True, rescale=True)

    a = _sc_msgpass(hs, src, dst, ew)
    hs = _tc_layer(a, dinv_i, dinv_o, W2, b2.reshape(1, D),
                   gamma.reshape(1, D), beta.reshape(1, D),
                   elu=True, rescale=True)

    a = _sc_msgpass(hs, src, dst, ew)
    out = _tc_layer(a, dinv_i, dinv_o, W3, b3.reshape(1, D),
                    gamma.reshape(1, D), beta.reshape(1, D),
                    elu=False, rescale=False)
    return out


# speedup vs baseline: 6.1309x; 6.1309x over previous
import dataclasses
import functools

import jax
import jax.numpy as jnp
from jax import lax
from jax.experimental import pallas as pl
from jax.experimental.pallas import tpu as pltpu
from jax.experimental.pallas import tpu_sc as plsc

N = 10000
E = 320000
D = 128

NC = 2
NS = 16
NW = NC * NS
EPW = E // NW
K = 80
NCHUNK = EPW // K
GC = 25
NG = NCHUNK // GC
RPT = N // NS
LANES = 16

_mesh = plsc.VectorSubcoreMesh(core_axis_name="c", subcore_axis_name="s")


_cp = pltpu.CompilerParams()
if "needs_layout_passes" in pltpu.CompilerParams.__dataclass_fields__:
    _cp = dataclasses.replace(_cp, needs_layout_passes=False)


@functools.partial(
    pl.kernel,
    out_type=jax.ShapeDtypeStruct((NW * 2 * N,), jnp.float32),
    mesh=_mesh,
    compiler_params=_cp,
    scratch_types=[
        pltpu.VMEM((EPW,), jnp.int32),
        pltpu.VMEM((EPW,), jnp.int32),
        pltpu.VMEM((2 * N,), jnp.float32),
    ],
)
def _sc_degrees(src_hbm, dst_hbm, out_hbm, sbuf, dbuf, hist):
    core = lax.axis_index("c")
    sid = lax.axis_index("s")
    wid = sid * NC + core

    @pl.loop(0, 2 * N // LANES)
    def _(i):
        hist[pl.ds(i * LANES, LANES)] = jnp.zeros((LANES,), jnp.float32)

    pltpu.sync_copy(src_hbm.at[pl.ds(wid * EPW, EPW)], sbuf)
    pltpu.sync_copy(dst_hbm.at[pl.ds(wid * EPW, EPW)], dbuf)

    ones16 = jnp.full((LANES,), 1.0, jnp.float32)

    @pl.loop(0, EPW // LANES)
    def _(j):
        iv = sbuf[pl.ds(j * LANES, LANES)]
        plsc.addupdate_scatter(hist, [iv], ones16)
        iv2 = dbuf[pl.ds(j * LANES, LANES)] + N
        plsc.addupdate_scatter(hist, [iv2], ones16)

    pltpu.sync_copy(hist, out_hbm.at[pl.ds(wid * 2 * N, 2 * N)])


@functools.partial(
    pl.kernel,
    out_type=jax.ShapeDtypeStruct((NC, N, D), jnp.float32),
    mesh=_mesh,
    scratch_types=[
        pltpu.VMEM((GC, K), jnp.int32),
        pltpu.VMEM((GC, K), jnp.int32),
        pltpu.VMEM((GC, K), jnp.float32),
        pltpu.VMEM((K, D), jnp.float32),
        pltpu.VMEM_SHARED((N, D), jnp.float32),
        pltpu.SemaphoreType.DMA,
    ],
)
def _sc_msgpass(hs_hbm, src_hbm, dst_hbm, ew_hbm, out_hbm,
                sbuf, dbuf, wbuf, rows, acc, sem):
    core = lax.axis_index("c")
    sid = lax.axis_index("s")
    wid = sid * NC + core

    @pl.loop(0, K)
    def _(i):
        for j in range(D // LANES):
            rows[i, pl.ds(j * LANES, LANES)] = jnp.zeros((LANES,), jnp.float32)

    @pl.loop(0, RPT // K)
    def _(k):
        pltpu.sync_copy(rows, acc.at[pl.ds(sid * RPT + k * K, K)])

    rem = RPT % K
    pltpu.sync_copy(rows.at[pl.ds(0, rem)],
                    acc.at[pl.ds(sid * RPT + (RPT // K) * K, rem)])

    plsc.subcore_barrier()

    @pl.loop(0, NG)
    def _(g):
        pltpu.sync_copy(src_hbm.at[wid, g], sbuf)
        pltpu.sync_copy(dst_hbm.at[wid, g], dbuf)
        pltpu.sync_copy(ew_hbm.at[wid, g], wbuf)

        @pl.loop(0, GC)
        def _(c):
            pltpu.async_copy(hs_hbm.at[sbuf.at[c]], rows, sem).wait()

            @pl.loop(0, K // LANES)
            def _(k):
                w16 = wbuf[c, pl.ds(k * LANES, LANES)]
                for i in range(LANES):
                    bv = jnp.broadcast_to(w16[i], (LANES,))
                    row = k * LANES + i
                    for j in range(D // LANES):
                        slc = (row, pl.ds(j * LANES, LANES))
                        rows[slc] = rows[slc] * bv

            pltpu.sync_copy(rows, acc.at[dbuf.at[c]], add=True)

    plsc.subcore_barrier()

    @pl.when(sid < NS - 1)
    def _():
        pltpu.sync_copy(acc.at[pl.ds(sid * 624, 624)],
                        out_hbm.at[core, pl.ds(sid * 624, 624)])

    @pl.when(sid == NS - 1)
    def _():
        pltpu.sync_copy(acc.at[pl.ds((NS - 1) * 624, 640)],
                        out_hbm.at[core, pl.ds((NS - 1) * 624, 640)])


def _tc_prep_body(degs, x, dinv_o, dinv_i, hs1):
    ones_w = jnp.ones((NW, 1), jnp.float32)
    deg2 = lax.dot_general(degs[...], ones_w, (((0,), (0,)), ((), ())),
                           preferred_element_type=jnp.float32)
    deg_o = jnp.maximum(deg2[:N], 1.0)
    deg_i = jnp.maximum(deg2[N:], 1.0)
    io = lax.rsqrt(deg_o)
    ii = lax.rsqrt(deg_i)
    dinv_o[...] = io
    dinv_i[...] = ii
    hs1[...] = x[...] * io


def _tc_prep(degs, x):
    return pl.pallas_call(
        _tc_prep_body,
        out_shape=(
            jax.ShapeDtypeStruct((N, 1), jnp.float32),
            jax.ShapeDtypeStruct((N, 1), jnp.float32),
            jax.ShapeDtypeStruct((N, D), jnp.float32),
        ),
    )(degs, x)


def _tc_layer_body(a, dinv_i, dinv_o, W, b, gamma, beta, out, *, elu, rescale):
    t = (a[0] + a[1]) * dinv_i[...]
    y = jnp.dot(t, W[...], preferred_element_type=jnp.float32) + b[...]
    mean = jnp.mean(y, axis=0, keepdims=True)
    var = jnp.mean((y - mean) ** 2, axis=0, keepdims=True)
    z = (y - mean) * lax.rsqrt(var + 1e-5) * gamma[...] + beta[...]
    if elu:
        z = jnp.where(z > 0, z, jnp.exp(jnp.minimum(z, 0.0)) - 1.0)
    if rescale:
        z = z * dinv_o[...]
    out[...] = z


def _tc_layer(a, dinv_i, dinv_o, W, b, gamma, beta, *, elu, rescale):
    body = functools.partial(_tc_layer_body, elu=elu, rescale=rescale)
    return pl.pallas_call(
        body,
        out_shape=jax.ShapeDtypeStruct((N, D), jnp.float32),
    )(a, dinv_i, dinv_o, W, b, gamma, beta)


def kernel(node_weight, edge_index, edge_weight, W1, b1, W2, b2, W3, b3,
           gamma, beta):
    src = edge_index[0].reshape(NW, NG, GC, K)
    dst = edge_index[1].reshape(NW, NG, GC, K)
    ew = edge_weight.reshape(NW, NG, GC, K)

    degs = _sc_degrees(edge_index[0], edge_index[1]).reshape(NW, 2 * N)
    dinv_o, dinv_i, hs = _tc_prep(degs, node_weight)

    a = _sc_msgpass(hs, src, dst, ew)
    hs = _tc_layer(a, dinv_i, dinv_o, W1, b1.reshape(1, D),
                   gamma.reshape(1, D), beta.reshape(1, D),
                   elu=True, rescale=True)

    a = _sc_msgpass(hs, src, dst, ew)
    hs = _tc_layer(a, dinv_i, dinv_o, W2, b2.reshape(1, D),
                   gamma.reshape(1, D), beta.reshape(1, D),
                   elu=True, rescale=True)

    a = _sc_msgpass(hs, src, dst, ew)
    out = _tc_layer(a, dinv_i, dinv_o, W3, b3.reshape(1, D),
                    gamma.reshape(1, D), beta.reshape(1, D),
                    elu=False, rescale=False)
    return out


# speedup vs baseline: 8.9884x; 1.4661x over previous
import dataclasses
import functools

import jax
import jax.numpy as jnp
from jax import lax
from jax.experimental import pallas as pl
from jax.experimental.pallas import tpu as pltpu
from jax.experimental.pallas import tpu_sc as plsc

N = 10000
E = 320000
D = 128

NC = 2
NS = 16
NW = NC * NS
EPW = E // NW
K = 80
NCHUNK = EPW // K
GC = 25
NG = NCHUNK // GC
RPT = N // NS
LANES = 16

_mesh = plsc.VectorSubcoreMesh(core_axis_name="c", subcore_axis_name="s")


_cp = pltpu.CompilerParams()
if "needs_layout_passes" in pltpu.CompilerParams.__dataclass_fields__:
    _cp = dataclasses.replace(_cp, needs_layout_passes=False)


@functools.partial(
    pl.kernel,
    out_type=jax.ShapeDtypeStruct((NW * 2 * N,), jnp.float32),
    mesh=_mesh,
    compiler_params=_cp,
    scratch_types=[
        pltpu.VMEM((EPW,), jnp.int32),
        pltpu.VMEM((EPW,), jnp.int32),
        pltpu.VMEM((2 * N,), jnp.float32),
    ],
)
def _sc_degrees(src_hbm, dst_hbm, out_hbm, sbuf, dbuf, hist):
    core = lax.axis_index("c")
    sid = lax.axis_index("s")
    wid = sid * NC + core

    @pl.loop(0, 2 * N // LANES)
    def _(i):
        hist[pl.ds(i * LANES, LANES)] = jnp.zeros((LANES,), jnp.float32)

    pltpu.sync_copy(src_hbm.at[pl.ds(wid * EPW, EPW)], sbuf)
    pltpu.sync_copy(dst_hbm.at[pl.ds(wid * EPW, EPW)], dbuf)

    ones16 = jnp.full((LANES,), 1.0, jnp.float32)

    @pl.loop(0, EPW // LANES)
    def _(j):
        iv = sbuf[pl.ds(j * LANES, LANES)]
        plsc.addupdate_scatter(hist, [iv], ones16)
        iv2 = dbuf[pl.ds(j * LANES, LANES)] + N
        plsc.addupdate_scatter(hist, [iv2], ones16)

    pltpu.sync_copy(hist, out_hbm.at[pl.ds(wid * 2 * N, 2 * N)])


@functools.partial(
    pl.kernel,
    out_type=jax.ShapeDtypeStruct((NC, N, D), jnp.float32),
    mesh=_mesh,
    scratch_types=[
        pltpu.VMEM((GC, K), jnp.int32),
        pltpu.VMEM((GC, K), jnp.int32),
        pltpu.VMEM((GC, K), jnp.float32),
        pltpu.VMEM((GC, K), jnp.int32),
        pltpu.VMEM((GC, K), jnp.int32),
        pltpu.VMEM((GC, K), jnp.float32),
        pltpu.VMEM((K, D), jnp.float32),
        pltpu.VMEM((K, D), jnp.float32),
        pltpu.VMEM_SHARED((N, D), jnp.float32),
        pltpu.SemaphoreType.DMA,
        pltpu.SemaphoreType.DMA,
        pltpu.SemaphoreType.DMA,
        pltpu.SemaphoreType.DMA,
        pltpu.SemaphoreType.DMA,
    ],
)
def _sc_msgpass(hs_hbm, src_hbm, dst_hbm, ew_hbm, out_hbm,
                sA, dA, wA, sB, dB, wB, r0, r1, acc,
                g0, g1, s0, s1, isem):
    core = lax.axis_index("c")
    sid = lax.axis_index("s")
    wid = sid * NC + core

    @pl.loop(0, K)
    def _(i):
        for j in range(D // LANES):
            r0[i, pl.ds(j * LANES, LANES)] = jnp.zeros((LANES,), jnp.float32)

    @pl.loop(0, RPT // K)
    def _(k):
        pltpu.sync_copy(r0, acc.at[pl.ds(sid * RPT + k * K, K)])

    rem = RPT % K
    pltpu.sync_copy(r0.at[pl.ds(0, rem)],
                    acc.at[pl.ds(sid * RPT + (RPT // K) * K, rem)])

    plsc.subcore_barrier()

    def scale(rbuf, wref, c):
        @pl.loop(0, K // LANES)
        def _(k):
            w16 = wref[c, pl.ds(k * LANES, LANES)]
            for i in range(LANES):
                bv = jnp.broadcast_to(w16[i], (LANES,))
                row = k * LANES + i
                for j in range(D // LANES):
                    slc = (row, pl.ds(j * LANES, LANES))
                    rbuf[slc] = rbuf[slc] * bv

    pltpu.sync_copy(src_hbm.at[wid, 0], sA)
    pltpu.sync_copy(dst_hbm.at[wid, 0], dA)
    pltpu.sync_copy(ew_hbm.at[wid, 0], wA)

    NP = (GC - 3) // 2

    for g in range(NG):
        sX, dX, wX = (sA, dA, wA) if g % 2 == 0 else (sB, dB, wB)
        sY, dY, wY = (sB, dB, wB) if g % 2 == 0 else (sA, dA, wA)

        if g > 0:
            pltpu.make_async_copy(src_hbm.at[wid, g], sX, isem).wait()
            pltpu.make_async_copy(dst_hbm.at[wid, g], dX, isem).wait()
            pltpu.make_async_copy(ew_hbm.at[wid, g], wX, isem).wait()

        pltpu.async_copy(hs_hbm.at[sX.at[0]], r0, g0)
        pltpu.async_copy(hs_hbm.at[sX.at[1]], r1, g1)

        if g + 1 < NG:
            pltpu.async_copy(src_hbm.at[wid, g + 1], sY, isem)
            pltpu.async_copy(dst_hbm.at[wid, g + 1], dY, isem)
            pltpu.async_copy(ew_hbm.at[wid, g + 1], wY, isem)

        @pl.loop(0, NP)
        def _(q):
            c0 = 2 * q
            pltpu.make_async_copy(hs_hbm.at[sX.at[0]], r0, g0).wait()
            scale(r0, wX, c0)
            pltpu.async_copy(r0, acc.at[dX.at[c0]], s0, add=True)
            pltpu.make_async_copy(hs_hbm.at[sX.at[0]], r1, g1).wait()
            scale(r1, wX, c0 + 1)
            pltpu.async_copy(r1, acc.at[dX.at[c0 + 1]], s1, add=True)
            pltpu.make_async_copy(r0, acc.at[dX.at[0]], s0).wait()
            pltpu.async_copy(hs_hbm.at[sX.at[c0 + 2]], r0, g0)
            pltpu.make_async_copy(r1, acc.at[dX.at[0]], s1).wait()
            pltpu.async_copy(hs_hbm.at[sX.at[c0 + 3]], r1, g1)

        cE = 2 * NP
        pltpu.make_async_copy(hs_hbm.at[sX.at[0]], r0, g0).wait()
        scale(r0, wX, cE)
        pltpu.async_copy(r0, acc.at[dX.at[cE]], s0, add=True)
        pltpu.make_async_copy(hs_hbm.at[sX.at[0]], r1, g1).wait()
        scale(r1, wX, cE + 1)
        pltpu.async_copy(r1, acc.at[dX.at[cE + 1]], s1, add=True)
        pltpu.make_async_copy(r0, acc.at[dX.at[0]], s0).wait()
        pltpu.async_copy(hs_hbm.at[sX.at[cE + 2]], r0, g0)
        pltpu.make_async_copy(hs_hbm.at[sX.at[0]], r0, g0).wait()
        scale(r0, wX, cE + 2)
        pltpu.async_copy(r0, acc.at[dX.at[cE + 2]], s0, add=True)
        pltpu.make_async_copy(r0, acc.at[dX.at[0]], s0).wait()
        pltpu.make_async_copy(r1, acc.at[dX.at[0]], s1).wait()

    plsc.subcore_barrier()

    @pl.when(sid < NS - 1)
    def _():
        pltpu.sync_copy(acc.at[pl.ds(sid * 624, 624)],
                        out_hbm.at[core, pl.ds(sid * 624, 624)])

    @pl.when(sid == NS - 1)
    def _():
        pltpu.sync_copy(acc.at[pl.ds((NS - 1) * 624, 640)],
                        out_hbm.at[core, pl.ds((NS - 1) * 624, 640)])


def _tc_prep_body(degs, x, dinv_o, dinv_i, hs1):
    ones_w = jnp.ones((NW, 1), jnp.float32)
    deg2 = lax.dot_general(degs[...], ones_w, (((0,), (0,)), ((), ())),
                           preferred_element_type=jnp.float32)
    deg_o = jnp.maximum(deg2[:N], 1.0)
    deg_i = jnp.maximum(deg2[N:], 1.0)
    io = lax.rsqrt(deg_o)
    ii = lax.rsqrt(deg_i)
    dinv_o[...] = io
    dinv_i[...] = ii
    hs1[...] = x[...] * io


def _tc_prep(degs, x):
    return pl.pallas_call(
        _tc_prep_body,
        out_shape=(
            jax.ShapeDtypeStruct((N, 1), jnp.float32),
            jax.ShapeDtypeStruct((N, 1), jnp.float32),
            jax.ShapeDtypeStruct((N, D), jnp.float32),
        ),
    )(degs, x)


def _tc_layer_body(a, dinv_i, dinv_o, W, b, gamma, beta, out, *, elu, rescale):
    t = (a[0] + a[1]) * dinv_i[...]
    y = jnp.dot(t, W[...], preferred_element_type=jnp.float32) + b[...]
    mean = jnp.mean(y, axis=0, keepdims=True)
    var = jnp.mean((y - mean) ** 2, axis=0, keepdims=True)
    z = (y - mean) * lax.rsqrt(var + 1e-5) * gamma[...] + beta[...]
    if elu:
        z = jnp.where(z > 0, z, jnp.exp(jnp.minimum(z, 0.0)) - 1.0)
    if rescale:
        z = z * dinv_o[...]
    out[...] = z


def _tc_layer(a, dinv_i, dinv_o, W, b, gamma, beta, *, elu, rescale):
    body = functools.partial(_tc_layer_body, elu=elu, rescale=rescale)
    return pl.pallas_call(
        body,
        out_shape=jax.ShapeDtypeStruct((N, D), jnp.float32),
    )(a, dinv_i, dinv_o, W, b, gamma, beta)


def kernel(node_weight, edge_index, edge_weight, W1, b1, W2, b2, W3, b3,
           gamma, beta):
    src = edge_index[0].reshape(NW, NG, GC, K)
    dst = edge_index[1].reshape(NW, NG, GC, K)
    ew = edge_weight.reshape(NW, NG, GC, K)

    degs = _sc_degrees(edge_index[0], edge_index[1]).reshape(NW, 2 * N)
    dinv_o, dinv_i, hs = _tc_prep(degs, node_weight)

    a = _sc_msgpass(hs, src, dst, ew)
    hs = _tc_layer(a, dinv_i, dinv_o, W1, b1.reshape(1, D),
                   gamma.reshape(1, D), beta.reshape(1, D),
                   elu=True, rescale=True)

    a = _sc_msgpass(hs, src, dst, ew)
    hs = _tc_layer(a, dinv_i, dinv_o, W2, b2.reshape(1, D),
                   gamma.reshape(1, D), beta.reshape(1, D),
                   elu=True, rescale=True)

    a = _sc_msgpass(hs, src, dst, ew)
    out = _tc_layer(a, dinv_i, dinv_o, W3, b3.reshape(1, D),
                    gamma.reshape(1, D), beta.reshape(1, D),
                    elu=False, rescale=False)
    return out


# speedup vs baseline: 10.4168x; 1.1589x over previous
import dataclasses
import functools

import jax
import jax.numpy as jnp
from jax import lax
from jax.experimental import pallas as pl
from jax.experimental.pallas import tpu as pltpu
from jax.experimental.pallas import tpu_sc as plsc

N = 10000
E = 320000
D = 128

NC = 2
NS = 16
NW = NC * NS
EPW = E // NW
K = 80
NCHUNK = EPW // K
GC = 25
NG = NCHUNK // GC
RPT = N // NS
LANES = 16

_mesh = plsc.VectorSubcoreMesh(core_axis_name="c", subcore_axis_name="s")


_cp = pltpu.CompilerParams()
if "needs_layout_passes" in pltpu.CompilerParams.__dataclass_fields__:
    _cp = dataclasses.replace(_cp, needs_layout_passes=False)


@functools.partial(
    pl.kernel,
    out_type=jax.ShapeDtypeStruct((NW * 2 * N,), jnp.float32),
    mesh=_mesh,
    compiler_params=_cp,
    scratch_types=[
        pltpu.VMEM((EPW,), jnp.int32),
        pltpu.VMEM((EPW,), jnp.int32),
        pltpu.VMEM((2 * N,), jnp.float32),
    ],
)
def _sc_degrees(src_hbm, dst_hbm, out_hbm, sbuf, dbuf, hist):
    core = lax.axis_index("c")
    sid = lax.axis_index("s")
    wid = sid * NC + core

    @pl.loop(0, 2 * N // LANES)
    def _(i):
        hist[pl.ds(i * LANES, LANES)] = jnp.zeros((LANES,), jnp.float32)

    pltpu.sync_copy(src_hbm.at[pl.ds(wid * EPW, EPW)], sbuf)
    pltpu.sync_copy(dst_hbm.at[pl.ds(wid * EPW, EPW)], dbuf)

    ones16 = jnp.full((LANES,), 1.0, jnp.float32)

    @pl.loop(0, EPW // LANES)
    def _(j):
        iv = sbuf[pl.ds(j * LANES, LANES)]
        plsc.addupdate_scatter(hist, [iv], ones16)
        iv2 = dbuf[pl.ds(j * LANES, LANES)] + N
        plsc.addupdate_scatter(hist, [iv2], ones16)

    pltpu.sync_copy(hist, out_hbm.at[pl.ds(wid * 2 * N, 2 * N)])


@functools.partial(
    pl.kernel,
    out_type=jax.ShapeDtypeStruct((NC, N, D), jnp.float32),
    mesh=_mesh,
    scratch_types=[
        pltpu.VMEM((2 * GC, K), jnp.int32),
        pltpu.VMEM((GC, K), jnp.float32),
        pltpu.VMEM((K, D), jnp.float32),
        pltpu.VMEM((K, D), jnp.float32),
        pltpu.VMEM((K, D), jnp.float32),
        pltpu.VMEM_SHARED((N, D), jnp.float32),
        pltpu.SemaphoreType.DMA,
        pltpu.SemaphoreType.DMA,
        pltpu.SemaphoreType.DMA,
        pltpu.SemaphoreType.DMA,
        pltpu.SemaphoreType.DMA,
        pltpu.SemaphoreType.DMA,
    ],
)
def _sc_msgpass(hs_hbm, sd_hbm, ew_hbm, out_hbm,
                i2, wbuf, r0, r1, r2, acc, g0, g1, g2, s0, s1, s2):
    core = lax.axis_index("c")
    sid = lax.axis_index("s")
    wid = sid * NC + core

    @pl.loop(0, K)
    def _(i):
        for j in range(D // LANES):
            r0[i, pl.ds(j * LANES, LANES)] = jnp.zeros((LANES,), jnp.float32)

    @pl.loop(0, RPT // K)
    def _(k):
        pltpu.sync_copy(r0, acc.at[pl.ds(sid * RPT + k * K, K)])

    rem = RPT % K
    pltpu.sync_copy(r0.at[pl.ds(0, rem)],
                    acc.at[pl.ds(sid * RPT + (RPT // K) * K, rem)])

    plsc.subcore_barrier()

    def scale(rbuf, c):
        @pl.loop(0, K // LANES)
        def _(k):
            w16 = wbuf[c, pl.ds(k * LANES, LANES)]
            for i in range(LANES):
                bv = jnp.broadcast_to(w16[i], (LANES,))
                row = k * LANES + i
                for j in range(D // LANES):
                    slc = (row, pl.ds(j * LANES, LANES))
                    rbuf[slc] = rbuf[slc] * bv

    B = [(r0, g0, s0), (r1, g1, s1), (r2, g2, s2)]

    def wait_gather(bt):
        pltpu.make_async_copy(hs_hbm.at[i2.at[0]], bt[0], bt[1]).wait()

    def issue_gather(bt, c):
        pltpu.async_copy(hs_hbm.at[i2.at[c]], bt[0], bt[1])

    def issue_scatter(bt, c):
        pltpu.async_copy(bt[0], acc.at[i2.at[GC + c]], bt[2], add=True)

    def wait_scatter(bt):
        pltpu.make_async_copy(bt[0], acc.at[i2.at[GC]], bt[2]).wait()

    def body(m_bt, p_bt, c, cg=None):
        wait_gather(m_bt)
        scale(m_bt[0], c)
        issue_scatter(m_bt, c)
        if cg is not None:
            wait_scatter(p_bt)
            issue_gather(p_bt, cg)

    for g in range(NG):
        pltpu.sync_copy(sd_hbm.at[wid, g], i2)
        pltpu.sync_copy(ew_hbm.at[wid, g], wbuf)

        issue_gather(B[0], 0)
        issue_gather(B[1], 1)

        wait_gather(B[0])
        scale(r0, 0)
        issue_scatter(B[0], 0)
        issue_gather(B[2], 2)

        @pl.loop(0, 7)
        def _(t):
            m = 3 * t + 1
            body(B[1], B[0], m, m + 2)
            body(B[2], B[1], m + 1, m + 3)
            body(B[0], B[2], m + 2, m + 4)

        body(B[1], B[0], 22, 24)
        body(B[2], B[1], 23, None)
        wait_scatter(B[1])
        body(B[0], B[2], 24, None)
        wait_scatter(B[2])
        wait_scatter(B[0])

    plsc.subcore_barrier()

    @pl.when(sid < NS - 1)
    def _():
        pltpu.sync_copy(acc.at[pl.ds(sid * 624, 624)],
                        out_hbm.at[core, pl.ds(sid * 624, 624)])

    @pl.when(sid == NS - 1)
    def _():
        pltpu.sync_copy(acc.at[pl.ds((NS - 1) * 624, 640)],
                        out_hbm.at[core, pl.ds((NS - 1) * 624, 640)])


def _tc_prep_body(degs, x, dinv_o, dinv_i, hs1):
    ones_w = jnp.ones((NW, 1), jnp.float32)
    deg2 = lax.dot_general(degs[...], ones_w, (((0,), (0,)), ((), ())),
                           preferred_element_type=jnp.float32)
    deg_o = jnp.maximum(deg2[:N], 1.0)
    deg_i = jnp.maximum(deg2[N:], 1.0)
    io = lax.rsqrt(deg_o)
    ii = lax.rsqrt(deg_i)
    dinv_o[...] = io
    dinv_i[...] = ii
    hs1[...] = x[...] * io


def _tc_prep(degs, x):
    return pl.pallas_call(
        _tc_prep_body,
        out_shape=(
            jax.ShapeDtypeStruct((N, 1), jnp.float32),
            jax.ShapeDtypeStruct((N, 1), jnp.float32),
            jax.ShapeDtypeStruct((N, D), jnp.float32),
        ),
    )(degs, x)


def _tc_layer_body(a, dinv_i, dinv_o, W, b, gamma, beta, out, *, elu, rescale):
    t = (a[0] + a[1]) * dinv_i[...]
    y = jnp.dot(t, W[...], preferred_element_type=jnp.float32) + b[...]
    mean = jnp.mean(y, axis=0, keepdims=True)
    var = jnp.mean((y - mean) ** 2, axis=0, keepdims=True)
    z = (y - mean) * lax.rsqrt(var + 1e-5) * gamma[...] + beta[...]
    if elu:
        z = jnp.where(z > 0, z, jnp.exp(jnp.minimum(z, 0.0)) - 1.0)
    if rescale:
        z = z * dinv_o[...]
    out[...] = z


def _tc_layer(a, dinv_i, dinv_o, W, b, gamma, beta, *, elu, rescale):
    body = functools.partial(_tc_layer_body, elu=elu, rescale=rescale)
    return pl.pallas_call(
        body,
        out_shape=jax.ShapeDtypeStruct((N, D), jnp.float32),
    )(a, dinv_i, dinv_o, W, b, gamma, beta)


def kernel(node_weight, edge_index, edge_weight, W1, b1, W2, b2, W3, b3,
           gamma, beta):
    src = edge_index[0].reshape(NW, NG, GC, K)
    dst = edge_index[1].reshape(NW, NG, GC, K)
    sd = jnp.concatenate([src, dst], axis=2)
    ew = edge_weight.reshape(NW, NG, GC, K)

    degs = _sc_degrees(edge_index[0], edge_index[1]).reshape(NW, 2 * N)
    dinv_o, dinv_i, hs = _tc_prep(degs, node_weight)

    a = _sc_msgpass(hs, sd, ew)
    hs = _tc_layer(a, dinv_i, dinv_o, W1, b1.reshape(1, D),
                   gamma.reshape(1, D), beta.reshape(1, D),
                   elu=True, rescale=True)

    a = _sc_msgpass(hs, sd, ew)
    hs = _tc_layer(a, dinv_i, dinv_o, W2, b2.reshape(1, D),
                   gamma.reshape(1, D), beta.reshape(1, D),
                   elu=True, rescale=True)

    a = _sc_msgpass(hs, sd, ew)
    out = _tc_layer(a, dinv_i, dinv_o, W3, b3.reshape(1, D),
                    gamma.reshape(1, D), beta.reshape(1, D),
                    elu=False, rescale=False)
    return out
